# baseline (speedup 1.0000x reference)
import functools

import jax
import jax.numpy as jnp
from jax import lax
from jax.experimental import pallas as pl
from jax.experimental.pallas import tpu as pltpu
from jax.experimental.pallas import tpu_sc as plsc

B, T, D = 8, 2048, 1024
GD = 64
W = 16
NWIN = T // W
TGT = 1024
NEG = -3.0e38

TB = 512
SDEPTH = 6


def _sim_body(z_hbm, w1t_ref, w2t_ref, e_ref, zbuf, sems):
    nj = T // TB
    nstep = pl.num_programs(0) * nj
    step = pl.program_id(0) * nj + pl.program_id(1)

    def start_fetch(k, slot):
        pltpu.make_async_copy(
            z_hbm.at[k // nj, pl.ds((k % nj) * TB, TB), :],
            zbuf.at[slot], sems.at[slot],
        ).start()

    @pl.when(step == 0)
    def _():
        for k in range(SDEPTH - 1):
            start_fetch(k, k)

    @pl.when(step + SDEPTH - 1 < nstep)
    def _():
        start_fetch(step + SDEPTH - 1, (step + SDEPTH - 1) % SDEPTH)

    slot = step % SDEPTH
    pltpu.make_async_copy(
        z_hbm.at[0, pl.ds(0, TB), :], zbuf.at[slot], sems.at[slot]
    ).wait()

    zb = zbuf[slot]
    h = jax.lax.dot_general(zb, w1t_ref[...], (((1,), (0,)), ((), ())),
                            preferred_element_type=jnp.float32)
    h = jnp.maximum(h, 0.0)
    g = jax.lax.dot_general(h, w2t_ref[...], (((1,), (0,)), ((), ())),
                            preferred_element_type=jnp.float32)
    nrm = jnp.sqrt(jnp.sum(g * g, axis=1, keepdims=True))
    g = g / (nrm + 1e-8)
    gp = jnp.concatenate([g[:1], g[:-1]], axis=0)
    d = jnp.sum(g * gp, axis=1)
    i = jax.lax.broadcasted_iota(jnp.int32, (TB,), 0)
    e_ref[0, 0, 0, :] = jnp.where(i % W == 0, NEG, d)


def _make_sim(bn):
    return pl.pallas_call(
        _sim_body,
        grid=(bn, T // TB),
        in_specs=[
            pl.BlockSpec(memory_space=pl.ANY),
            pl.BlockSpec((D, GD), lambda b, j: (0, 0)),
            pl.BlockSpec((GD, GD), lambda b, j: (0, 0)),
        ],
        out_specs=pl.BlockSpec((1, 1, 1, TB), lambda b, j: (b, j, 0, 0)),
        out_shape=jax.ShapeDtypeStruct((bn, T // TB, 1, TB), jnp.float32),
        scratch_shapes=[
            pltpu.VMEM((SDEPTH, TB, D), jnp.float32),
            pltpu.SemaphoreType.DMA((SDEPTH,)),
        ],
    )

@functools.cache
def _build_merge_sc(bn):
    mesh = plsc.VectorSubcoreMesh(core_axis_name="c", subcore_axis_name="s")
    return functools.partial(
        pl.kernel,
        out_type=[
            jax.ShapeDtypeStruct((bn, TGT), jnp.int32),
            jax.ShapeDtypeStruct((bn, TGT), jnp.int32),
        ],
        mesh=mesh,
        compiler_params=pltpu.CompilerParams(needs_layout_passes=False),
        scratch_types=[
            pltpu.VMEM((T,), jnp.float32),
            pltpu.VMEM((T + 32,), jnp.int32),
            pltpu.VMEM((TGT,), jnp.int32),
            pltpu.VMEM((96,), jnp.float32),
            pltpu.VMEM((96,), jnp.int32),
            pltpu.VMEM((120,), jnp.int32),
        ],
    )(functools.partial(_merge_sc_body, bn))


def _merge_sc_body(bn, e_hbm, idx_hbm, lens_hbm, e_v, idxs_v, lens_v, kbuf,
                   pbuf, ibuf):
    wid = lax.axis_index("s") * 2 + lax.axis_index("c")

    @pl.when(wid < bn)
    def _():
        pltpu.sync_copy(e_hbm.at[wid], e_v)
        iota = jnp.arange(16, dtype=jnp.int32)
        lane0 = iota == 0
        zv = jnp.zeros((16,), jnp.int32)
        negv = jnp.full((16,), NEG, jnp.float32)
        for off in (0, 64):
            kbuf[pl.ds(off, 16)] = negv
            kbuf[pl.ds(off + 16, 16)] = negv
            pbuf[pl.ds(off, 16)] = zv
            pbuf[pl.ds(off + 16, 16)] = zv
            ibuf[pl.ds(off, 16)] = zv
            ibuf[pl.ds(off + 32, 16)] = zv

        def halfbody(w, off):
            key0 = e_v[pl.ds(w * 16, 16)]
            alive = jnp.logical_not(lane0)
            picked = jnp.zeros((16,), jnp.bool_)
            for _ in range(8):
                keyc = jnp.where(alive, key0, NEG)
                kbuf[pl.ds(off + 1, 16)] = keyc
                kl = kbuf[pl.ds(off, 16)]
                kr = kbuf[pl.ds(off + 2, 16)]
                p = alive & (keyc > kl) & (keyc >= kr)
                picked = picked | p
                pbuf[pl.ds(off + 1, 16)] = jnp.where(p, 1, 0).astype(jnp.int32)
                pn = (pbuf[pl.ds(off, 16)] + pbuf[pl.ds(off + 2, 16)]) > 0
                alive = alive & jnp.logical_not(p) & jnp.logical_not(pn)
            keep = jnp.logical_not(picked)
            k32 = jnp.where(keep, 1, 0).astype(jnp.int32)
            x = k32
            for k in (1, 2, 4, 8):
                ibuf[pl.ds(off + 16, 16)] = x
                x = x + ibuf[pl.ds(off + 16 - k, 16)]
            y = k32
            for k in (1, 2, 4, 8):
                ibuf[pl.ds(off + 16, 16)] = y
                y = y + ibuf[pl.ds(off + 16 + k, 16)]
            tot = (x + y) - k32
            return x, k32, tot, keep

        def wbody(w, cntv):
            xa, ka, tota, keepa = halfbody(2 * w, 0)
            xb, kb, totb, keepb = halfbody(2 * w + 1, 64)
            plsc.store_scatter(idxs_v, [(cntv + xa) - ka],
                               2 * w * 16 + iota, mask=keepa)
            cnt2 = cntv + tota
            plsc.store_scatter(idxs_v, [(cnt2 + xb) - kb],
                               (2 * w + 1) * 16 + iota, mask=keepb)
            return cnt2 + totb

        cnt = lax.fori_loop(0, NWIN // 2, wbody, jnp.zeros((16,), jnp.int32))
        plsc.store_scatter(idxs_v, [cnt],
                           jnp.full((16,), T, jnp.int32), mask=lane0)

        def lbody(i, c):
            a = idxs_v[pl.ds(i * 16, 16)]
            nx = idxs_v[pl.ds(i * 16 + 1, 16)]
            lens_v[pl.ds(i * 16, 16)] = nx - a
            return c

        lax.fori_loop(0, TGT // 16, lbody, jnp.int32(0))
        pltpu.sync_copy(idxs_v.at[pl.ds(0, TGT)], idx_hbm.at[wid])
        pltpu.sync_copy(lens_v, lens_hbm.at[wid])


G = 256
SPAN = 2 * G + 8
NSTEP = B * (TGT // G)


def _gather_body(s_ref, z_hbm, idxv_ref, lensv_ref, out_ref, zbuf, sems):
    step = pl.program_id(0) * (TGT // G) + pl.program_id(1)

    def start_fetch(k, slot):
        bb = k // (TGT // G)
        jj = k % (TGT // G)
        st = jnp.minimum((s_ref[bb * TGT + jj * G] // 8) * 8, T - SPAN)
        pltpu.make_async_copy(
            z_hbm.at[bb, pl.ds(st, SPAN), :], zbuf.at[slot], sems.at[slot]
        ).start()

    @pl.when(step == 0)
    def _():
        start_fetch(0, 0)
        start_fetch(1, 1)
        start_fetch(2, 2)

    @pl.when(step + 3 < NSTEP)
    def _():
        start_fetch(step + 3, (step + 3) % 4)

    slot = step % 4
    pltpu.make_async_copy(
        z_hbm.at[0, pl.ds(0, SPAN), :], zbuf.at[slot], sems.at[slot]
    ).wait()

    idxs = idxv_ref[0, 0, :]
    lens = lensv_ref[0, 0, :]
    st0 = jnp.minimum((idxv_ref[0, 0, 0] // 8) * 8, T - SPAN)
    loc = idxs - st0
    w0 = jnp.where(lens == 2, jnp.float32(0.5), jnp.float32(1.0))
    w1 = jnp.where(lens == 2, jnp.float32(0.5), jnp.float32(0.0))
    locc = loc[:, None]
    c = jax.lax.broadcasted_iota(jnp.int32, (G, SPAN), 1)
    smat = (jnp.where(c == locc, w0[:, None], 0.0)
            + jnp.where(c == locc + 1, w1[:, None], 0.0))
    out_ref[0] = jax.lax.dot_general(
        smat, zbuf[slot], (((1,), (0,)), ((), ())),
        preferred_element_type=jnp.float32)


_gather_grid = pltpu.PrefetchScalarGridSpec(
    num_scalar_prefetch=1,
    grid=(B, TGT // G),
    in_specs=[
        pl.BlockSpec(memory_space=pl.ANY),
        pl.BlockSpec((1, 1, G), lambda b, j, s: (b * (TGT // G) + j, 0, 0)),
        pl.BlockSpec((1, 1, G), lambda b, j, s: (b * (TGT // G) + j, 0, 0)),
    ],
    out_specs=pl.BlockSpec((1, G, D), lambda b, j, s: (b, j, 0)),
    scratch_shapes=[
        pltpu.VMEM((4, SPAN, D), jnp.float32),
        pltpu.SemaphoreType.DMA((4,)),
    ],
)

_gather_call = pl.pallas_call(
    _gather_body,
    grid_spec=_gather_grid,
    out_shape=jax.ShapeDtypeStruct((B, TGT, D), jnp.float32),
)


def kernel(z, token_lens, target_len, W1, W2):
    e = _make_sim(B)(z, W1.T, W2.T).reshape(B, T)
    idx, lens = _build_merge_sc(B)(e)
    idx3 = idx.reshape(B * (TGT // G), 1, G)
    lens3 = lens.reshape(B * (TGT // G), 1, G)
    z_new = _gather_call(idx.reshape(-1), z, idx3, lens3)
    return (z_new, lens, idx)


# speedup vs baseline: 1.0458x; 1.0458x over previous
import functools

import jax
import jax.numpy as jnp
from jax import lax
from jax.experimental import pallas as pl
from jax.experimental.pallas import tpu as pltpu
from jax.experimental.pallas import tpu_sc as plsc

B, T, D = 8, 2048, 1024
GD = 64
W = 16
NWIN = T // W
TGT = 1024
NEG = -3.0e38

TB = 2048
SDEPTH = 3


def _sim_body(z_hbm, w1t_ref, w2t_ref, e_ref, zbuf, sems):
    nj = T // TB
    nstep = pl.num_programs(0) * nj
    step = pl.program_id(0) * nj + pl.program_id(1)

    def start_fetch(k, slot):
        pltpu.make_async_copy(
            z_hbm.at[k // nj, pl.ds((k % nj) * TB, TB), :],
            zbuf.at[slot], sems.at[slot],
        ).start()

    @pl.when(step == 0)
    def _():
        for k in range(SDEPTH - 1):
            start_fetch(k, k)

    @pl.when(step + SDEPTH - 1 < nstep)
    def _():
        start_fetch(step + SDEPTH - 1, (step + SDEPTH - 1) % SDEPTH)

    slot = step % SDEPTH
    pltpu.make_async_copy(
        z_hbm.at[0, pl.ds(0, TB), :], zbuf.at[slot], sems.at[slot]
    ).wait()

    zb = zbuf[slot]
    h = jax.lax.dot_general(zb, w1t_ref[...], (((1,), (0,)), ((), ())),
                            preferred_element_type=jnp.float32)
    h = jnp.maximum(h, 0.0)
    g = jax.lax.dot_general(h, w2t_ref[...], (((1,), (0,)), ((), ())),
                            preferred_element_type=jnp.float32)
    nrm = jnp.sqrt(jnp.sum(g * g, axis=1, keepdims=True))
    g = g / (nrm + 1e-8)
    gp = jnp.concatenate([g[:1], g[:-1]], axis=0)
    d = jnp.sum(g * gp, axis=1)
    i = jax.lax.broadcasted_iota(jnp.int32, (TB,), 0)
    e_ref[0, 0, 0, :] = jnp.where(i % W == 0, NEG, d)


def _make_sim(bn):
    return pl.pallas_call(
        _sim_body,
        grid=(bn, T // TB),
        in_specs=[
            pl.BlockSpec(memory_space=pl.ANY),
            pl.BlockSpec((D, GD), lambda b, j: (0, 0)),
            pl.BlockSpec((GD, GD), lambda b, j: (0, 0)),
        ],
        out_specs=pl.BlockSpec((1, 1, 1, TB), lambda b, j: (b, j, 0, 0)),
        out_shape=jax.ShapeDtypeStruct((bn, T // TB, 1, TB), jnp.float32),
        scratch_shapes=[
            pltpu.VMEM((SDEPTH, TB, D), jnp.float32),
            pltpu.SemaphoreType.DMA((SDEPTH,)),
        ],
    )

@functools.cache
def _build_merge_sc(bn):
    mesh = plsc.VectorSubcoreMesh(core_axis_name="c", subcore_axis_name="s")
    return functools.partial(
        pl.kernel,
        out_type=[
            jax.ShapeDtypeStruct((bn, TGT), jnp.int32),
            jax.ShapeDtypeStruct((bn, TGT), jnp.int32),
        ],
        mesh=mesh,
        compiler_params=pltpu.CompilerParams(needs_layout_passes=False),
        scratch_types=[
            pltpu.VMEM((T,), jnp.float32),
            pltpu.VMEM((T + 32,), jnp.int32),
            pltpu.VMEM((TGT,), jnp.int32),
            pltpu.VMEM((96,), jnp.float32),
            pltpu.VMEM((96,), jnp.int32),
            pltpu.VMEM((120,), jnp.int32),
        ],
    )(functools.partial(_merge_sc_body, bn))


def _merge_sc_body(bn, e_hbm, idx_hbm, lens_hbm, e_v, idxs_v, lens_v, kbuf,
                   pbuf, ibuf):
    wid = lax.axis_index("s") * 2 + lax.axis_index("c")

    @pl.when(wid < bn)
    def _():
        pltpu.sync_copy(e_hbm.at[wid], e_v)
        iota = jnp.arange(16, dtype=jnp.int32)
        lane0 = iota == 0
        zv = jnp.zeros((16,), jnp.int32)
        negv = jnp.full((16,), NEG, jnp.float32)
        for off in (0, 64):
            kbuf[pl.ds(off, 16)] = negv
            kbuf[pl.ds(off + 16, 16)] = negv
            pbuf[pl.ds(off, 16)] = zv
            pbuf[pl.ds(off + 16, 16)] = zv
            ibuf[pl.ds(off, 16)] = zv
            ibuf[pl.ds(off + 32, 16)] = zv

        def halfbody(w, off):
            key0 = e_v[pl.ds(w * 16, 16)]
            alive = jnp.logical_not(lane0)
            picked = jnp.zeros((16,), jnp.bool_)
            for _ in range(8):
                keyc = jnp.where(alive, key0, NEG)
                kbuf[pl.ds(off + 1, 16)] = keyc
                kl = kbuf[pl.ds(off, 16)]
                kr = kbuf[pl.ds(off + 2, 16)]
                p = alive & (keyc > kl) & (keyc >= kr)
                picked = picked | p
                pbuf[pl.ds(off + 1, 16)] = jnp.where(p, 1, 0).astype(jnp.int32)
                pn = (pbuf[pl.ds(off, 16)] + pbuf[pl.ds(off + 2, 16)]) > 0
                alive = alive & jnp.logical_not(p) & jnp.logical_not(pn)
            keep = jnp.logical_not(picked)
            k32 = jnp.where(keep, 1, 0).astype(jnp.int32)
            x = k32
            for k in (1, 2, 4, 8):
                ibuf[pl.ds(off + 16, 16)] = x
                x = x + ibuf[pl.ds(off + 16 - k, 16)]
            y = k32
            for k in (1, 2, 4, 8):
                ibuf[pl.ds(off + 16, 16)] = y
                y = y + ibuf[pl.ds(off + 16 + k, 16)]
            tot = (x + y) - k32
            return x, k32, tot, keep

        def wbody(w, cntv):
            xa, ka, tota, keepa = halfbody(2 * w, 0)
            xb, kb, totb, keepb = halfbody(2 * w + 1, 64)
            plsc.store_scatter(idxs_v, [(cntv + xa) - ka],
                               2 * w * 16 + iota, mask=keepa)
            cnt2 = cntv + tota
            plsc.store_scatter(idxs_v, [(cnt2 + xb) - kb],
                               (2 * w + 1) * 16 + iota, mask=keepb)
            return cnt2 + totb

        cnt = lax.fori_loop(0, NWIN // 2, wbody, jnp.zeros((16,), jnp.int32))
        plsc.store_scatter(idxs_v, [cnt],
                           jnp.full((16,), T, jnp.int32), mask=lane0)

        def lbody(i, c):
            a = idxs_v[pl.ds(i * 16, 16)]
            nx = idxs_v[pl.ds(i * 16 + 1, 16)]
            lens_v[pl.ds(i * 16, 16)] = nx - a
            return c

        lax.fori_loop(0, TGT // 16, lbody, jnp.int32(0))
        pltpu.sync_copy(idxs_v.at[pl.ds(0, TGT)], idx_hbm.at[wid])
        pltpu.sync_copy(lens_v, lens_hbm.at[wid])


G = 256
SPAN = 2 * G + 8
NSTEP = B * (TGT // G)


def _gather_body(s_ref, z_hbm, idxv_ref, lensv_ref, out_ref, zbuf, sems):
    step = pl.program_id(0) * (TGT // G) + pl.program_id(1)

    def start_fetch(k, slot):
        bb = k // (TGT // G)
        jj = k % (TGT // G)
        st = jnp.minimum((s_ref[bb * TGT + jj * G] // 8) * 8, T - SPAN)
        pltpu.make_async_copy(
            z_hbm.at[bb, pl.ds(st, SPAN), :], zbuf.at[slot], sems.at[slot]
        ).start()

    @pl.when(step == 0)
    def _():
        start_fetch(0, 0)
        start_fetch(1, 1)
        start_fetch(2, 2)

    @pl.when(step + 3 < NSTEP)
    def _():
        start_fetch(step + 3, (step + 3) % 4)

    slot = step % 4
    pltpu.make_async_copy(
        z_hbm.at[0, pl.ds(0, SPAN), :], zbuf.at[slot], sems.at[slot]
    ).wait()

    idxs = idxv_ref[0, 0, :]
    lens = lensv_ref[0, 0, :]
    st0 = jnp.minimum((idxv_ref[0, 0, 0] // 8) * 8, T - SPAN)
    loc = idxs - st0
    w0 = jnp.where(lens == 2, jnp.float32(0.5), jnp.float32(1.0))
    w1 = jnp.where(lens == 2, jnp.float32(0.5), jnp.float32(0.0))
    locc = loc[:, None]
    c = jax.lax.broadcasted_iota(jnp.int32, (G, SPAN), 1)
    smat = (jnp.where(c == locc, w0[:, None], 0.0)
            + jnp.where(c == locc + 1, w1[:, None], 0.0))
    out_ref[0] = jax.lax.dot_general(
        smat, zbuf[slot], (((1,), (0,)), ((), ())),
        preferred_element_type=jnp.float32)


_gather_grid = pltpu.PrefetchScalarGridSpec(
    num_scalar_prefetch=1,
    grid=(B, TGT // G),
    in_specs=[
        pl.BlockSpec(memory_space=pl.ANY),
        pl.BlockSpec((1, 1, G), lambda b, j, s: (b * (TGT // G) + j, 0, 0)),
        pl.BlockSpec((1, 1, G), lambda b, j, s: (b * (TGT // G) + j, 0, 0)),
    ],
    out_specs=pl.BlockSpec((1, G, D), lambda b, j, s: (b, j, 0)),
    scratch_shapes=[
        pltpu.VMEM((4, SPAN, D), jnp.float32),
        pltpu.SemaphoreType.DMA((4,)),
    ],
)

_gather_call = pl.pallas_call(
    _gather_body,
    grid_spec=_gather_grid,
    out_shape=jax.ShapeDtypeStruct((B, TGT, D), jnp.float32),
)


def kernel(z, token_lens, target_len, W1, W2):
    e = _make_sim(B)(z, W1.T, W2.T).reshape(B, T)
    idx, lens = _build_merge_sc(B)(e)
    idx3 = idx.reshape(B * (TGT // G), 1, G)
    lens3 = lens.reshape(B * (TGT // G), 1, G)
    z_new = _gather_call(idx.reshape(-1), z, idx3, lens3)
    return (z_new, lens, idx)


# speedup vs baseline: 1.0484x; 1.0025x over previous
import functools

import jax
import jax.numpy as jnp
from jax import lax
from jax.experimental import pallas as pl
from jax.experimental.pallas import tpu as pltpu
from jax.experimental.pallas import tpu_sc as plsc

B, T, D = 8, 2048, 1024
GD = 64
W = 16
NWIN = T // W
TGT = 1024
NEG = -3.0e38

TB = 1024
SDEPTH = 4


def _sim_body(z_hbm, w1t_ref, w2t_ref, e_ref, zbuf, sems):
    nj = T // TB
    nstep = pl.num_programs(0) * nj
    step = pl.program_id(0) * nj + pl.program_id(1)

    def start_fetch(k, slot):
        pltpu.make_async_copy(
            z_hbm.at[k // nj, pl.ds((k % nj) * TB, TB), :],
            zbuf.at[slot], sems.at[slot],
        ).start()

    @pl.when(step == 0)
    def _():
        for k in range(SDEPTH - 1):
            start_fetch(k, k)

    @pl.when(step + SDEPTH - 1 < nstep)
    def _():
        start_fetch(step + SDEPTH - 1, (step + SDEPTH - 1) % SDEPTH)

    slot = step % SDEPTH
    pltpu.make_async_copy(
        z_hbm.at[0, pl.ds(0, TB), :], zbuf.at[slot], sems.at[slot]
    ).wait()

    zb = zbuf[slot]
    h = jax.lax.dot_general(zb, w1t_ref[...], (((1,), (0,)), ((), ())),
                            preferred_element_type=jnp.float32)
    h = jnp.maximum(h, 0.0)
    g = jax.lax.dot_general(h, w2t_ref[...], (((1,), (0,)), ((), ())),
                            preferred_element_type=jnp.float32)
    nrm = jnp.sqrt(jnp.sum(g * g, axis=1, keepdims=True))
    g = g / (nrm + 1e-8)
    gp = jnp.concatenate([g[:1], g[:-1]], axis=0)
    d = jnp.sum(g * gp, axis=1)
    i = jax.lax.broadcasted_iota(jnp.int32, (TB,), 0)
    e_ref[0, 0, 0, :] = jnp.where(i % W == 0, NEG, d)


def _make_sim(bn):
    return pl.pallas_call(
        _sim_body,
        grid=(bn, T // TB),
        in_specs=[
            pl.BlockSpec(memory_space=pl.ANY),
            pl.BlockSpec((D, GD), lambda b, j: (0, 0)),
            pl.BlockSpec((GD, GD), lambda b, j: (0, 0)),
        ],
        out_specs=pl.BlockSpec((1, 1, 1, TB), lambda b, j: (b, j, 0, 0)),
        out_shape=jax.ShapeDtypeStruct((bn, T // TB, 1, TB), jnp.float32),
        scratch_shapes=[
            pltpu.VMEM((SDEPTH, TB, D), jnp.float32),
            pltpu.SemaphoreType.DMA((SDEPTH,)),
        ],
    )

@functools.cache
def _build_merge_sc(bn):
    mesh = plsc.VectorSubcoreMesh(core_axis_name="c", subcore_axis_name="s")
    return functools.partial(
        pl.kernel,
        out_type=[
            jax.ShapeDtypeStruct((bn, TGT), jnp.int32),
            jax.ShapeDtypeStruct((bn, TGT), jnp.int32),
        ],
        mesh=mesh,
        compiler_params=pltpu.CompilerParams(needs_layout_passes=False),
        scratch_types=[
            pltpu.VMEM((T,), jnp.float32),
            pltpu.VMEM((T + 32,), jnp.int32),
            pltpu.VMEM((TGT,), jnp.int32),
            pltpu.VMEM((96,), jnp.float32),
            pltpu.VMEM((96,), jnp.int32),
            pltpu.VMEM((120,), jnp.int32),
        ],
    )(functools.partial(_merge_sc_body, bn))


def _merge_sc_body(bn, e_hbm, idx_hbm, lens_hbm, e_v, idxs_v, lens_v, kbuf,
                   pbuf, ibuf):
    wid = lax.axis_index("s") * 2 + lax.axis_index("c")

    @pl.when(wid < bn)
    def _():
        pltpu.sync_copy(e_hbm.at[wid], e_v)
        iota = jnp.arange(16, dtype=jnp.int32)
        lane0 = iota == 0
        zv = jnp.zeros((16,), jnp.int32)
        negv = jnp.full((16,), NEG, jnp.float32)
        for off in (0, 64):
            kbuf[pl.ds(off, 16)] = negv
            kbuf[pl.ds(off + 16, 16)] = negv
            pbuf[pl.ds(off, 16)] = zv
            pbuf[pl.ds(off + 16, 16)] = zv
            ibuf[pl.ds(off, 16)] = zv
            ibuf[pl.ds(off + 32, 16)] = zv

        def halfbody(w, off):
            key0 = e_v[pl.ds(w * 16, 16)]
            alive = jnp.logical_not(lane0)
            picked = jnp.zeros((16,), jnp.bool_)
            for _ in range(8):
                keyc = jnp.where(alive, key0, NEG)
                kbuf[pl.ds(off + 1, 16)] = keyc
                kl = kbuf[pl.ds(off, 16)]
                kr = kbuf[pl.ds(off + 2, 16)]
                p = alive & (keyc > kl) & (keyc >= kr)
                picked = picked | p
                pbuf[pl.ds(off + 1, 16)] = jnp.where(p, 1, 0).astype(jnp.int32)
                pn = (pbuf[pl.ds(off, 16)] + pbuf[pl.ds(off + 2, 16)]) > 0
                alive = alive & jnp.logical_not(p) & jnp.logical_not(pn)
            keep = jnp.logical_not(picked)
            k32 = jnp.where(keep, 1, 0).astype(jnp.int32)
            x = k32
            for k in (1, 2, 4, 8):
                ibuf[pl.ds(off + 16, 16)] = x
                x = x + ibuf[pl.ds(off + 16 - k, 16)]
            y = k32
            for k in (1, 2, 4, 8):
                ibuf[pl.ds(off + 16, 16)] = y
                y = y + ibuf[pl.ds(off + 16 + k, 16)]
            tot = (x + y) - k32
            return x, k32, tot, keep

        def wbody(w, cntv):
            xa, ka, tota, keepa = halfbody(2 * w, 0)
            xb, kb, totb, keepb = halfbody(2 * w + 1, 64)
            plsc.store_scatter(idxs_v, [(cntv + xa) - ka],
                               2 * w * 16 + iota, mask=keepa)
            cnt2 = cntv + tota
            plsc.store_scatter(idxs_v, [(cnt2 + xb) - kb],
                               (2 * w + 1) * 16 + iota, mask=keepb)
            return cnt2 + totb

        cnt = lax.fori_loop(0, NWIN // 2, wbody, jnp.zeros((16,), jnp.int32))
        plsc.store_scatter(idxs_v, [cnt],
                           jnp.full((16,), T, jnp.int32), mask=lane0)

        def lbody(i, c):
            a = idxs_v[pl.ds(i * 16, 16)]
            nx = idxs_v[pl.ds(i * 16 + 1, 16)]
            lens_v[pl.ds(i * 16, 16)] = nx - a
            return c

        lax.fori_loop(0, TGT // 16, lbody, jnp.int32(0))
        pltpu.sync_copy(idxs_v.at[pl.ds(0, TGT)], idx_hbm.at[wid])
        pltpu.sync_copy(lens_v, lens_hbm.at[wid])


G = 512
SPAN = 2 * G + 8
NSTEP = B * (TGT // G)


def _gather_body(s_ref, z_hbm, idxv_ref, lensv_ref, out_ref, zbuf, sems):
    step = pl.program_id(0) * (TGT // G) + pl.program_id(1)

    def start_fetch(k, slot):
        bb = k // (TGT // G)
        jj = k % (TGT // G)
        st = jnp.minimum((s_ref[bb * TGT + jj * G] // 8) * 8, T - SPAN)
        pltpu.make_async_copy(
            z_hbm.at[bb, pl.ds(st, SPAN), :], zbuf.at[slot], sems.at[slot]
        ).start()

    @pl.when(step == 0)
    def _():
        start_fetch(0, 0)
        start_fetch(1, 1)
        start_fetch(2, 2)

    @pl.when(step + 3 < NSTEP)
    def _():
        start_fetch(step + 3, (step + 3) % 4)

    slot = step % 4
    pltpu.make_async_copy(
        z_hbm.at[0, pl.ds(0, SPAN), :], zbuf.at[slot], sems.at[slot]
    ).wait()

    idxs = idxv_ref[0, 0, :]
    lens = lensv_ref[0, 0, :]
    st0 = jnp.minimum((idxv_ref[0, 0, 0] // 8) * 8, T - SPAN)
    loc = idxs - st0
    w0 = jnp.where(lens == 2, jnp.float32(0.5), jnp.float32(1.0))
    w1 = jnp.where(lens == 2, jnp.float32(0.5), jnp.float32(0.0))
    locc = loc[:, None]
    c = jax.lax.broadcasted_iota(jnp.int32, (G, SPAN), 1)
    smat = (jnp.where(c == locc, w0[:, None], 0.0)
            + jnp.where(c == locc + 1, w1[:, None], 0.0))
    out_ref[0] = jax.lax.dot_general(
        smat, zbuf[slot], (((1,), (0,)), ((), ())),
        preferred_element_type=jnp.float32)


_gather_grid = pltpu.PrefetchScalarGridSpec(
    num_scalar_prefetch=1,
    grid=(B, TGT // G),
    in_specs=[
        pl.BlockSpec(memory_space=pl.ANY),
        pl.BlockSpec((1, 1, G), lambda b, j, s: (b * (TGT // G) + j, 0, 0)),
        pl.BlockSpec((1, 1, G), lambda b, j, s: (b * (TGT // G) + j, 0, 0)),
    ],
    out_specs=pl.BlockSpec((1, G, D), lambda b, j, s: (b, j, 0)),
    scratch_shapes=[
        pltpu.VMEM((4, SPAN, D), jnp.float32),
        pltpu.SemaphoreType.DMA((4,)),
    ],
)

_gather_call = pl.pallas_call(
    _gather_body,
    grid_spec=_gather_grid,
    out_shape=jax.ShapeDtypeStruct((B, TGT, D), jnp.float32),
)


def kernel(z, token_lens, target_len, W1, W2):
    e = _make_sim(B)(z, W1.T, W2.T).reshape(B, T)
    idx, lens = _build_merge_sc(B)(e)
    idx3 = idx.reshape(B * (TGT // G), 1, G)
    lens3 = lens.reshape(B * (TGT // G), 1, G)
    z_new = _gather_call(idx.reshape(-1), z, idx3, lens3)
    return (z_new, lens, idx)
